# baseline (speedup 1.0000x reference)
import functools
import math

import jax
import jax.numpy as jnp
from jax import lax
from jax.experimental import pallas as pl
from jax.experimental.pallas import tpu as pltpu
from jax.experimental.pallas import tpu_sc as plsc

D_MODEL = 64
D_PAD = 128
SCALE = math.sqrt(D_MODEL)

NUM_CORES = 2
NUM_SUBCORES = 16
NUM_WORKERS = NUM_CORES * NUM_SUBCORES
LANES = 16

TBLK = 128


@functools.lru_cache(maxsize=None)
def _build_format(vocab: int):
    nblk = vocab // TBLK
    tail = vocab - nblk * TBLK
    per_w = nblk // NUM_WORKERS
    extra = nblk - per_w * NUM_WORKERS
    pairs = per_w // 2
    assert per_w % 2 == 0 and vocab >= 2 * TBLK

    mesh = plsc.VectorSubcoreMesh(core_axis_name="c", subcore_axis_name="s")

    @functools.partial(
        pl.kernel,
        mesh=mesh,
        compiler_params=pltpu.CompilerParams(needs_layout_passes=False),
        out_type=jax.ShapeDtypeStruct((vocab, D_PAD), jnp.float32),
        scratch_types=[
            pltpu.VMEM((D_MODEL, TBLK), jnp.float32),
            pltpu.VMEM((D_MODEL, TBLK), jnp.float32),
            pltpu.VMEM((TBLK, D_PAD), jnp.float32),
            pltpu.VMEM((TBLK, D_PAD), jnp.float32),
            pltpu.VMEM((max(tail, 8), D_MODEL), jnp.float32),
            pltpu.SemaphoreType.DMA,
            pltpu.SemaphoreType.DMA,
            pltpu.SemaphoreType.DMA,
            pltpu.SemaphoreType.DMA,
        ],
    )
    def fmt_kernel(lut_t, lut_tail, tbl, sb0, sb1, db0, db1, stb,
                   is0, is1, os0, os1):
        wid = lax.axis_index("s") * NUM_CORES + lax.axis_index("c")

        cidx = [
            lax.iota(jnp.int32, LANES) + q * LANES
            for q in range(D_MODEL // LANES)
        ]

        def start_in(r0, sb, isem):
            pltpu.async_copy(lut_t.at[:, pl.ds(r0, TBLK)], sb, isem)

        def wait_in(sb, isem):
            pltpu.make_async_copy(lut_t.at[:, pl.ds(0, TBLK)], sb, isem).wait()

        def start_out(r0, db, osem, rows=TBLK):
            pltpu.async_copy(db.at[pl.ds(0, rows)], tbl.at[pl.ds(r0, rows)], osem)

        def wait_out(db, osem, rows=TBLK):
            pltpu.make_async_copy(
                db.at[pl.ds(0, rows)], tbl.at[pl.ds(0, rows)], osem
            ).wait()

        def compute(sb, db, rows=TBLK, col_off=0):
            def body(i, carry):
                col = jnp.full((LANES,), i + col_off, dtype=jnp.int32)
                for q in range(D_MODEL // LANES):
                    v = plsc.load_gather(sb, [cidx[q], col])
                    db[i, pl.ds(q * LANES, LANES)] = v * SCALE
                return carry

            lax.fori_loop(0, rows, body, 0, unroll=8)

        def blk0(k):
            return (k * NUM_WORKERS + wid) * TBLK

        start_in(blk0(0), sb0, is0)

        def pair(p, carry):
            k0 = 2 * p
            k1 = k0 + 1
            start_in(blk0(k1), sb1, is1)
            wait_in(sb0, is0)

            @pl.when(p > 0)
            def _():
                wait_out(db0, os0)

            compute(sb0, db0)
            start_out(blk0(k0), db0, os0)

            @pl.when(k1 + 1 < per_w)
            def _():
                start_in(blk0(k1 + 1), sb0, is0)

            wait_in(sb1, is1)

            @pl.when(p > 0)
            def _():
                wait_out(db1, os1)

            compute(sb1, db1)
            start_out(blk0(k1), db1, os1)
            return carry

        lax.fori_loop(0, pairs, pair, 0)
        wait_out(db0, os0)
        wait_out(db1, os1)

        @pl.when(wid < extra)
        def _():
            r0 = (per_w * NUM_WORKERS + wid) * TBLK
            start_in(r0, sb0, is0)
            wait_in(sb0, is0)
            compute(sb0, db0)
            start_out(r0, db0, os0)
            wait_out(db0, os0)

        if tail:
            @pl.when(wid == NUM_WORKERS - 1)
            def _():
                pltpu.sync_copy(lut_tail, stb)

                def body(i, carry):
                    for q in range(D_MODEL // LANES):
                        sl = pl.ds(q * LANES, LANES)
                        db0[i, sl] = stb[i, sl] * SCALE
                    return carry

                lax.fori_loop(0, tail, body, 0)
                start_out(vocab - tail, db0, os0, rows=tail)
                wait_out(db0, os0, rows=tail)

    return fmt_kernel


IDX_SPLITS = ((0, 128), (128, 72))


@functools.lru_cache(maxsize=None)
def _build_gather(batch: int, seq: int, vocab: int):
    rows_per_w = batch // NUM_WORKERS

    mesh = plsc.VectorSubcoreMesh(core_axis_name="c", subcore_axis_name="s")

    @functools.partial(
        pl.kernel,
        mesh=mesh,
        compiler_params=pltpu.CompilerParams(needs_layout_passes=False),
        out_type=jax.ShapeDtypeStruct((batch, seq, D_MODEL), jnp.float32),
        scratch_types=[
            pltpu.VMEM((rows_per_w, seq), jnp.int32),
            pltpu.VMEM((seq, D_PAD), jnp.float32),
            pltpu.VMEM((seq, D_PAD), jnp.float32),
            pltpu.VMEM((1, seq, D_MODEL), jnp.float32),
            pltpu.SemaphoreType.DMA,
            pltpu.SemaphoreType.DMA,
        ],
    )
    def gat_kernel(x_hbm, tbl_hbm, out_hbm, idx_v, buf0, buf1, cbuf,
                   sem0, sem1):
        wid = lax.axis_index("s") * NUM_CORES + lax.axis_index("c")
        row0 = wid * rows_per_w

        bufs = (buf0, buf1)
        sems = (sem0, sem1)

        pltpu.sync_copy(x_hbm.at[pl.ds(row0, rows_per_w)], idx_v)

        def fire(r, buf, sem):
            waits = []
            for off, n in IDX_SPLITS:
                waits.append(
                    pltpu.async_copy(
                        tbl_hbm.at[idx_v.at[r, pl.ds(off, n)]],
                        buf.at[pl.ds(off, n)],
                        sem,
                    )
                )
            return waits

        def compact(buf):
            def body(s, carry):
                for q in range(D_MODEL // LANES):
                    sl = pl.ds(q * LANES, LANES)
                    cbuf[0, s, sl] = buf[s, sl]
                return carry

            lax.fori_loop(0, seq, body, 0, unroll=8)

        inflight = fire(0, bufs[0], sems[0])
        for r in range(rows_per_w):
            cur = bufs[r % 2]
            nxt = (
                fire(r + 1, bufs[(r + 1) % 2], sems[(r + 1) % 2])
                if r + 1 < rows_per_w
                else []
            )
            for w in inflight:
                w.wait()
            inflight = nxt
            compact(cur)
            pltpu.sync_copy(cbuf, out_hbm.at[pl.ds(row0 + r, 1)])

    return gat_kernel


def kernel(x, lut):
    batch, seq = x.shape
    vocab = lut.shape[0]
    tail_rows = max(vocab % TBLK, 8)
    lut_tail = lax.slice(lut, (vocab - tail_rows, 0), (vocab, D_MODEL))
    tbl = _build_format(vocab)(jnp.transpose(lut), lut_tail)
    return _build_gather(batch, seq, vocab)(x.astype(jnp.int32), tbl)


# speedup vs baseline: 3.7134x; 3.7134x over previous
import functools
import math

import jax
import jax.numpy as jnp
from jax import lax
from jax.experimental import pallas as pl
from jax.experimental.pallas import tpu as pltpu
from jax.experimental.pallas import tpu_sc as plsc

D_MODEL = 64
SCALE = math.sqrt(D_MODEL)

NUM_CORES = 2
NUM_SUBCORES = 16
NUM_WORKERS = NUM_CORES * NUM_SUBCORES
LANES = 16

ROWS_PER_CHUNK = 2


@functools.lru_cache(maxsize=None)
def _build(batch: int, seq: int):
    rows_per_w = batch // NUM_WORKERS
    num_chunks = rows_per_w // ROWS_PER_CHUNK

    mesh = plsc.VectorSubcoreMesh(core_axis_name="c", subcore_axis_name="s")

    @functools.partial(
        pl.kernel,
        mesh=mesh,
        out_type=jax.ShapeDtypeStruct((batch, seq, D_MODEL), jnp.float32),
        scratch_types=[
            pltpu.VMEM((rows_per_w, seq), jnp.int32),
            pltpu.VMEM((ROWS_PER_CHUNK, seq, D_MODEL), jnp.float32),
            pltpu.VMEM((ROWS_PER_CHUNK, seq, D_MODEL), jnp.float32),
            pltpu.SemaphoreType.DMA,
            pltpu.SemaphoreType.DMA,
        ],
        compiler_params=pltpu.CompilerParams(use_tc_tiling_on_sc=True),
    )
    def emb_kernel(x_hbm, lut_hbm, out_hbm, idx_v, buf0, buf1, sem0, sem1):
        wid = lax.axis_index("s") * NUM_CORES + lax.axis_index("c")
        row0 = wid * rows_per_w

        bufs = (buf0, buf1)
        sems = (sem0, sem1)

        pltpu.sync_copy(x_hbm.at[pl.ds(row0, rows_per_w)], idx_v)

        def issue(g, buf, sem):
            def fetch16(r, s0, v, lanes):
                for j in lanes:
                    pltpu.async_copy(
                        lut_hbm.at[v[j]],
                        buf.at[r, s0 + j, pl.ds(0, D_MODEL)],
                        sem,
                    )

            for r in range(ROWS_PER_CHUNK):
                xrow = g * ROWS_PER_CHUNK + r

                def body(k, carry, r=r, xrow=xrow):
                    v = idx_v[xrow, pl.ds(k * LANES, LANES)]
                    fetch16(r, k * LANES, v, range(LANES))
                    return carry

                lax.fori_loop(0, seq // LANES, body, 0)
                tail = seq % LANES
                if tail:
                    v = idx_v[xrow, pl.ds(seq - LANES, LANES)]
                    fetch16(r, seq - LANES, v, range(LANES - tail, LANES))

        def drain(buf, sem):
            pltpu.make_async_copy(
                out_hbm.at[pl.ds(0, ROWS_PER_CHUNK)], buf, sem
            ).wait()

        def scale(buf):
            def body(s, carry):
                for r in range(ROWS_PER_CHUNK):
                    for c in range(D_MODEL // LANES):
                        sl = pl.ds(c * LANES, LANES)
                        buf[r, s, sl] = buf[r, s, sl] * SCALE
                return carry

            lax.fori_loop(0, seq, body, 0, unroll=4)

        issue(0, bufs[0], sems[0])
        for g in range(num_chunks):
            if g + 1 < num_chunks:
                issue(g + 1, bufs[(g + 1) % 2], sems[(g + 1) % 2])
            drain(bufs[g % 2], sems[g % 2])
            scale(bufs[g % 2])
            pltpu.sync_copy(
                bufs[g % 2],
                out_hbm.at[pl.ds(row0 + g * ROWS_PER_CHUNK, ROWS_PER_CHUNK)],
            )

    return emb_kernel


def kernel(x, lut):
    batch, seq = x.shape
    return _build(batch, seq)(x.astype(jnp.int32), lut)
